# baseline (speedup 1.0000x reference)
import functools

import numpy as np

import jax
import jax.numpy as jnp
from jax import lax
from jax.experimental import pallas as pl
from jax.experimental.pallas import tpu as pltpu
from jax.experimental.pallas import tpu_sc as plsc

N = 10000
D = 128
H = 128
E = 320000

NC = 2
NS = 16
NW = NC * NS
B = 128
DNW = 80
HH = H // 2
WB = 640
WR = WB // B
CWIN = 32
EPT = CWIN * WB
EPAD = NS * EPT
NPAD = 10240
ROWS_PT = NPAD // NS
DPT = NPAD // NS

_mesh = plsc.VectorSubcoreMesh(core_axis_name="c", subcore_axis_name="s")


@functools.partial(
    pl.kernel,
    mesh=_mesh,
    compiler_params=pltpu.CompilerParams(use_tc_tiling_on_sc=False),
    out_type=jax.ShapeDtypeStruct((NC * NPAD,), jnp.float32),
    scratch_types=[
        pltpu.VMEM_SHARED((NPAD,), jnp.float32),
        pltpu.VMEM((DNW, B), jnp.int32),
        pltpu.VMEM((B,), jnp.float32),
        pltpu.VMEM((DPT,), jnp.float32),
        pltpu.SemaphoreType.DMA,
    ],
)
def _deg_kernel(dst_hbm, out_hbm, acc, didx, ones, zbuf, sem):
    c = lax.axis_index("c")
    s = lax.axis_index("s")
    w = c * NS + s

    @pl.loop(0, B, step=16)
    def _(i):
        ones[pl.ds(i, 16)] = jnp.ones((16,), jnp.float32)

    @pl.loop(0, DPT, step=16)
    def _(i):
        zbuf[pl.ds(i, 16)] = jnp.zeros((16,), jnp.float32)

    pltpu.sync_copy(zbuf, acc.at[pl.ds(s * DPT, DPT)])
    pltpu.sync_copy(dst_hbm.at[pl.ds(w * DNW, DNW)], didx)
    plsc.subcore_barrier()

    @pl.loop(0, DNW)
    def _(win):
        pltpu.sync_copy(ones, acc.at[didx.at[win]], add=True)

    plsc.subcore_barrier()
    pltpu.sync_copy(acc.at[pl.ds(s * DPT, DPT)],
                    out_hbm.at[pl.ds(c * NPAD + s * DPT, DPT)])


def _scatter_win(rows, acc, didx, buf):
    for r in range(WR):
        pltpu.sync_copy(rows.at[pl.ds(r * B, B)], acc.at[didx.at[buf, r]],
                        add=True)


def _conv_core(y_hbm, s, srcf_hbm, dst3_hbm, zeros_hbm, acc,
               sidx0, sidx1, didx, rows0, rows1, sem0, sem1, si0, si1):
    sbase = s * EPT
    dbase = s * CWIN

    pltpu.sync_copy(srcf_hbm.at[pl.ds(sbase, WB)], sidx0)
    pltpu.sync_copy(dst3_hbm.at[dbase], didx.at[0])
    pltpu.async_copy(srcf_hbm.at[pl.ds(sbase + WB, WB)], sidx1, si1)
    pltpu.async_copy(dst3_hbm.at[dbase + 1], didx.at[1], si1)
    pltpu.async_copy(y_hbm.at[sidx0], rows0, sem0)

    pltpu.sync_copy(zeros_hbm.at[pl.ds(s * ROWS_PT, ROWS_PT)],
                    acc.at[pl.ds(s * ROWS_PT, ROWS_PT)])
    plsc.subcore_barrier()

    @pl.loop(0, CWIN, step=2)
    def _(w):
        pltpu.make_async_copy(srcf_hbm.at[pl.ds(sbase + (w + 1) * WB, WB)],
                              sidx1, si1).wait()
        pltpu.make_async_copy(dst3_hbm.at[dbase + w + 1], didx.at[1], si1).wait()
        pltpu.async_copy(y_hbm.at[sidx1], rows1, sem1)
        pltpu.make_async_copy(y_hbm.at[sidx0], rows0, sem0).wait()

        @pl.when(w + 2 < CWIN)
        def _():
            pltpu.async_copy(srcf_hbm.at[pl.ds(sbase + (w + 2) * WB, WB)],
                             sidx0, si0)

        _scatter_win(rows0, acc, didx, 0)

        @pl.when(w + 2 < CWIN)
        def _():
            pltpu.async_copy(dst3_hbm.at[dbase + w + 2], didx.at[0], si0)
            pltpu.make_async_copy(srcf_hbm.at[pl.ds(sbase + (w + 2) * WB, WB)],
                                  sidx0, si0).wait()
            pltpu.make_async_copy(dst3_hbm.at[dbase + w + 2], didx.at[0],
                                  si0).wait()
            pltpu.async_copy(y_hbm.at[sidx0], rows0, sem0)

        pltpu.make_async_copy(y_hbm.at[sidx1], rows1, sem1).wait()
        _scatter_win(rows1, acc, didx, 1)

        @pl.when(w + 3 < CWIN)
        def _():
            pltpu.async_copy(srcf_hbm.at[pl.ds(sbase + (w + 3) * WB, WB)],
                             sidx1, si1)
            pltpu.async_copy(dst3_hbm.at[dbase + w + 3], didx.at[1], si1)


@functools.partial(
    pl.kernel,
    mesh=_mesh,
    compiler_params=pltpu.CompilerParams(use_tc_tiling_on_sc=False),
    out_type=jax.ShapeDtypeStruct((NC, NPAD, HH), jnp.float32),
    scratch_types=[
        pltpu.VMEM_SHARED((NPAD, HH), jnp.float32),
        pltpu.VMEM((WB,), jnp.int32),
        pltpu.VMEM((WB,), jnp.int32),
        pltpu.VMEM((2, WR, B), jnp.int32),
        pltpu.VMEM((WB, HH), jnp.float32),
        pltpu.VMEM((WB, HH), jnp.float32),
        pltpu.SemaphoreType.DMA,
        pltpu.SemaphoreType.DMA,
        pltpu.SemaphoreType.DMA,
        pltpu.SemaphoreType.DMA,
    ],
)
def _conv_kernel(y0_hbm, y1_hbm, srcf_hbm, dst3_hbm, zeros_hbm, out_hbm, acc,
                 sidx0, sidx1, didx, rows0, rows1, sem0, sem1, si0, si1):
    c = lax.axis_index("c")
    s = lax.axis_index("s")

    @pl.when(c == 0)
    def _():
        _conv_core(y0_hbm, s, srcf_hbm, dst3_hbm, zeros_hbm, acc,
                   sidx0, sidx1, didx, rows0, rows1, sem0, sem1, si0, si1)

    @pl.when(c == 1)
    def _():
        _conv_core(y1_hbm, s, srcf_hbm, dst3_hbm, zeros_hbm, acc,
                   sidx0, sidx1, didx, rows0, rows1, sem0, sem1, si0, si1)

    plsc.subcore_barrier()
    pltpu.sync_copy(acc.at[pl.ds(s * ROWS_PT, ROWS_PT)],
                    out_hbm.at[c, pl.ds(s * ROWS_PT, ROWS_PT)])


def _lrelu(v):
    return jnp.where(v >= 0, v, 0.01 * v)


def _dinv_of(degt_ref):
    deg = degt_ref[:, 0] + degt_ref[:, 1] + 1.0
    return lax.rsqrt(deg)[:, None]


R = 2000


def _tc1_body(degt_ref, x_ref, w1_ref, y1a_ref, y1b_ref):
    xw = jnp.dot(x_ref[...], w1_ref[...], preferred_element_type=jnp.float32)
    y1 = xw * _dinv_of(degt_ref)
    y1a_ref[...] = y1[:, :HH]
    y1b_ref[...] = y1[:, HH:]


def _tc2_body(degt_ref, acc_ref, y1a_ref, y1b_ref, b1_ref, w2_ref,
              h1_ref, y2a_ref, y2b_ref):
    dinv = _dinv_of(degt_ref)
    accf = jnp.concatenate([acc_ref[0] + y1a_ref[...],
                            acc_ref[1] + y1b_ref[...]], axis=1)
    pre = dinv * accf + b1_ref[...]
    h1 = _lrelu(pre)
    h1_ref[...] = h1
    y2 = jnp.dot(h1, w2_ref[...], preferred_element_type=jnp.float32) * dinv
    y2a_ref[...] = y2[:, :HH]
    y2b_ref[...] = y2[:, HH:]


def _tc3_body(degt_ref, acc_ref, y2a_ref, y2b_ref, h1_ref, b2_ref,
              wm1_ref, bm1_ref, wm2_ref, bm2_ref,
              wp1_ref, bp1_ref, wp2_ref, bp2_ref, out_ref):
    dinv = _dinv_of(degt_ref)
    accf = jnp.concatenate([acc_ref[0] + y2a_ref[...],
                            acc_ref[1] + y2b_ref[...]], axis=1)
    g = dinv * accf + b2_ref[...]
    h2 = _lrelu(g + h1_ref[...])
    m = jnp.dot(_lrelu(jnp.dot(h2, wm1_ref[...], preferred_element_type=jnp.float32)
                       + bm1_ref[...]),
                wm2_ref[...], preferred_element_type=jnp.float32) + bm2_ref[...]
    h3 = _lrelu(m + h2)
    p = _lrelu(jnp.dot(h3, wp1_ref[...], preferred_element_type=jnp.float32) + bp1_ref[...])
    out_ref[...] = _lrelu(jnp.dot(p, wp2_ref[...], preferred_element_type=jnp.float32)
                          + bp2_ref[...])


def _row_spec(shape):
    nd = len(shape)
    if nd == 2:
        return pl.BlockSpec((R, shape[1]), lambda r: (r, 0))
    return pl.BlockSpec((shape[0], R, shape[2]), lambda r: (0, r, 0))


def _full_spec(shape):
    zeros = (0,) * len(shape)
    return pl.BlockSpec(shape, lambda r, _z=zeros: _z)


def _tc_call(body, in_arrays, row_flags, out_widths):
    in_specs = [
        _row_spec(a.shape) if rf else _full_spec(a.shape)
        for a, rf in zip(in_arrays, row_flags)
    ]
    out_shape = tuple(jax.ShapeDtypeStruct((N, wd), jnp.float32)
                      for wd in out_widths)
    out_specs = tuple(pl.BlockSpec((R, wd), lambda r, _w=wd: (r, 0))
                      for wd in out_widths)
    if len(out_widths) == 1:
        out_shape, out_specs = out_shape[0], out_specs[0]
    return pl.pallas_call(
        body,
        grid=(N // R,),
        in_specs=in_specs,
        out_specs=out_specs,
        out_shape=out_shape,
    )(*in_arrays)


def kernel(x, edge_index, W1, b1, W2, b2, Wm1, bm1, Wm2, bm2, Wp1, bp1, Wp2, bp2):
    npad_e = EPAD - E
    pad_src = jnp.asarray((np.arange(npad_e, dtype=np.int32) * 131) % N)
    pad_dst = jnp.asarray(N + (np.arange(npad_e, dtype=np.int32) % (NPAD - N)))
    src_flat = jnp.concatenate([edge_index[0], pad_src])
    dst_flat = jnp.concatenate([edge_index[1], pad_dst])
    dst3 = dst_flat.reshape(NS * CWIN, WR, B)
    dstw = dst_flat.reshape(EPAD // B, B)
    zeros = jnp.zeros((NPAD, HH), jnp.float32)

    degp = _deg_kernel(dstw)
    degt = degp.reshape(NC, NPAD).T

    y1a, y1b = _tc_call(_tc1_body, (degt, x, W1), (True, True, False), (HH, HH))
    a1 = _conv_kernel(y1a, y1b, src_flat, dst3, zeros)
    h1, y2a, y2b = _tc_call(_tc2_body, (degt, a1, y1a, y1b, b1[None], W2),
                            (True, True, True, True, False, False), (H, HH, HH))
    a2 = _conv_kernel(y2a, y2b, src_flat, dst3, zeros)
    p = _tc_call(_tc3_body,
                 (degt, a2, y2a, y2b, h1, b2[None], Wm1, bm1[None], Wm2,
                  bm2[None], Wp1, bp1[None], Wp2, bp2[None]),
                 (True, True, True, True, True) + (False,) * 9, (H,))
    return p


# speedup vs baseline: 1.0124x; 1.0124x over previous
import functools

import numpy as np

import jax
import jax.numpy as jnp
from jax import lax
from jax.experimental import pallas as pl
from jax.experimental.pallas import tpu as pltpu
from jax.experimental.pallas import tpu_sc as plsc

N = 10000
D = 128
H = 128
E = 320000

NC = 2
NS = 16
NW = NC * NS
B = 128
DGR = 16
HH = H // 2
WB = 640
WR = WB // B
CWIN = 32
EPT = CWIN * WB
EPAD = NS * EPT
NPAD = 10240
ROWS_PT = NPAD // NS
DPT = NPAD // NS

_mesh = plsc.VectorSubcoreMesh(core_axis_name="c", subcore_axis_name="s")


@functools.partial(
    pl.kernel,
    mesh=_mesh,
    compiler_params=pltpu.CompilerParams(use_tc_tiling_on_sc=False),
    out_type=jax.ShapeDtypeStruct((NC * NPAD,), jnp.float32),
    scratch_types=[
        pltpu.VMEM_SHARED((NPAD,), jnp.float32),
        pltpu.VMEM((DGR, WR, B), jnp.int32),
        pltpu.VMEM((B,), jnp.float32),
        pltpu.VMEM((DPT,), jnp.float32),
        pltpu.SemaphoreType.DMA,
    ],
)
def _deg_kernel(dst_hbm, out_hbm, acc, didx, ones, zbuf, sem):
    c = lax.axis_index("c")
    s = lax.axis_index("s")
    w = c * NS + s

    @pl.loop(0, B, step=16)
    def _(i):
        ones[pl.ds(i, 16)] = jnp.ones((16,), jnp.float32)

    @pl.loop(0, DPT, step=16)
    def _(i):
        zbuf[pl.ds(i, 16)] = jnp.zeros((16,), jnp.float32)

    pltpu.sync_copy(zbuf, acc.at[pl.ds(s * DPT, DPT)])
    pltpu.sync_copy(dst_hbm.at[pl.ds(w * DGR, DGR)], didx)
    plsc.subcore_barrier()

    @pl.loop(0, DGR)
    def _(g):
        for r in range(WR):
            pltpu.async_copy(ones, acc.at[didx.at[g, r]], sem, add=True)
        for r in range(WR):
            pltpu.make_async_copy(ones, acc.at[didx.at[g, r]], sem).wait()

    plsc.subcore_barrier()
    pltpu.sync_copy(acc.at[pl.ds(s * DPT, DPT)],
                    out_hbm.at[pl.ds(c * NPAD + s * DPT, DPT)])


def _scatter_win(rows, acc, didx, buf):
    for r in range(WR):
        pltpu.sync_copy(rows.at[pl.ds(r * B, B)], acc.at[didx.at[buf, r]],
                        add=True)


def _conv_core(y_hbm, s, srcf_hbm, dst3_hbm, zeros_hbm, acc,
               sidx0, sidx1, didx, rows0, rows1, sem0, sem1, si0, si1):
    sbase = s * EPT
    dbase = s * CWIN

    pltpu.sync_copy(srcf_hbm.at[pl.ds(sbase, WB)], sidx0)
    pltpu.sync_copy(dst3_hbm.at[dbase], didx.at[0])
    pltpu.async_copy(srcf_hbm.at[pl.ds(sbase + WB, WB)], sidx1, si1)
    pltpu.async_copy(dst3_hbm.at[dbase + 1], didx.at[1], si1)
    pltpu.async_copy(y_hbm.at[sidx0], rows0, sem0)

    pltpu.sync_copy(zeros_hbm.at[pl.ds(s * ROWS_PT, ROWS_PT)],
                    acc.at[pl.ds(s * ROWS_PT, ROWS_PT)])
    plsc.subcore_barrier()

    @pl.loop(0, CWIN, step=2)
    def _(w):
        pltpu.make_async_copy(srcf_hbm.at[pl.ds(sbase + (w + 1) * WB, WB)],
                              sidx1, si1).wait()
        pltpu.make_async_copy(dst3_hbm.at[dbase + w + 1], didx.at[1], si1).wait()
        pltpu.async_copy(y_hbm.at[sidx1], rows1, sem1)
        pltpu.make_async_copy(y_hbm.at[sidx0], rows0, sem0).wait()

        @pl.when(w + 2 < CWIN)
        def _():
            pltpu.async_copy(srcf_hbm.at[pl.ds(sbase + (w + 2) * WB, WB)],
                             sidx0, si0)

        _scatter_win(rows0, acc, didx, 0)

        @pl.when(w + 2 < CWIN)
        def _():
            pltpu.async_copy(dst3_hbm.at[dbase + w + 2], didx.at[0], si0)
            pltpu.make_async_copy(srcf_hbm.at[pl.ds(sbase + (w + 2) * WB, WB)],
                                  sidx0, si0).wait()
            pltpu.make_async_copy(dst3_hbm.at[dbase + w + 2], didx.at[0],
                                  si0).wait()
            pltpu.async_copy(y_hbm.at[sidx0], rows0, sem0)

        pltpu.make_async_copy(y_hbm.at[sidx1], rows1, sem1).wait()
        _scatter_win(rows1, acc, didx, 1)

        @pl.when(w + 3 < CWIN)
        def _():
            pltpu.async_copy(srcf_hbm.at[pl.ds(sbase + (w + 3) * WB, WB)],
                             sidx1, si1)
            pltpu.async_copy(dst3_hbm.at[dbase + w + 3], didx.at[1], si1)


@functools.partial(
    pl.kernel,
    mesh=_mesh,
    compiler_params=pltpu.CompilerParams(use_tc_tiling_on_sc=False),
    out_type=jax.ShapeDtypeStruct((NC, NPAD, HH), jnp.float32),
    scratch_types=[
        pltpu.VMEM_SHARED((NPAD, HH), jnp.float32),
        pltpu.VMEM((WB,), jnp.int32),
        pltpu.VMEM((WB,), jnp.int32),
        pltpu.VMEM((2, WR, B), jnp.int32),
        pltpu.VMEM((WB, HH), jnp.float32),
        pltpu.VMEM((WB, HH), jnp.float32),
        pltpu.SemaphoreType.DMA,
        pltpu.SemaphoreType.DMA,
        pltpu.SemaphoreType.DMA,
        pltpu.SemaphoreType.DMA,
    ],
)
def _conv_kernel(y0_hbm, y1_hbm, srcf_hbm, dst3_hbm, zeros_hbm, out_hbm, acc,
                 sidx0, sidx1, didx, rows0, rows1, sem0, sem1, si0, si1):
    c = lax.axis_index("c")
    s = lax.axis_index("s")

    @pl.when(c == 0)
    def _():
        _conv_core(y0_hbm, s, srcf_hbm, dst3_hbm, zeros_hbm, acc,
                   sidx0, sidx1, didx, rows0, rows1, sem0, sem1, si0, si1)

    @pl.when(c == 1)
    def _():
        _conv_core(y1_hbm, s, srcf_hbm, dst3_hbm, zeros_hbm, acc,
                   sidx0, sidx1, didx, rows0, rows1, sem0, sem1, si0, si1)

    plsc.subcore_barrier()
    pltpu.sync_copy(acc.at[pl.ds(s * ROWS_PT, ROWS_PT)],
                    out_hbm.at[c, pl.ds(s * ROWS_PT, ROWS_PT)])


def _lrelu(v):
    return jnp.where(v >= 0, v, 0.01 * v)


def _dinv_of(degt_ref):
    deg = degt_ref[:, 0] + degt_ref[:, 1] + 1.0
    return lax.rsqrt(deg)[:, None]


R = 2000


def _tc1_body(degt_ref, x_ref, w1_ref, y1a_ref, y1b_ref):
    xw = jnp.dot(x_ref[...], w1_ref[...], preferred_element_type=jnp.float32)
    y1 = xw * _dinv_of(degt_ref)
    y1a_ref[...] = y1[:, :HH]
    y1b_ref[...] = y1[:, HH:]


def _tc2_body(degt_ref, acc_ref, y1a_ref, y1b_ref, b1_ref, w2_ref,
              h1_ref, y2a_ref, y2b_ref):
    dinv = _dinv_of(degt_ref)
    accf = jnp.concatenate([acc_ref[0] + y1a_ref[...],
                            acc_ref[1] + y1b_ref[...]], axis=1)
    pre = dinv * accf + b1_ref[...]
    h1 = _lrelu(pre)
    h1_ref[...] = h1
    y2 = jnp.dot(h1, w2_ref[...], preferred_element_type=jnp.float32) * dinv
    y2a_ref[...] = y2[:, :HH]
    y2b_ref[...] = y2[:, HH:]


def _tc3_body(degt_ref, acc_ref, y2a_ref, y2b_ref, h1_ref, b2_ref,
              wm1_ref, bm1_ref, wm2_ref, bm2_ref,
              wp1_ref, bp1_ref, wp2_ref, bp2_ref, out_ref):
    dinv = _dinv_of(degt_ref)
    accf = jnp.concatenate([acc_ref[0] + y2a_ref[...],
                            acc_ref[1] + y2b_ref[...]], axis=1)
    g = dinv * accf + b2_ref[...]
    h2 = _lrelu(g + h1_ref[...])
    m = jnp.dot(_lrelu(jnp.dot(h2, wm1_ref[...], preferred_element_type=jnp.float32)
                       + bm1_ref[...]),
                wm2_ref[...], preferred_element_type=jnp.float32) + bm2_ref[...]
    h3 = _lrelu(m + h2)
    p = _lrelu(jnp.dot(h3, wp1_ref[...], preferred_element_type=jnp.float32) + bp1_ref[...])
    out_ref[...] = _lrelu(jnp.dot(p, wp2_ref[...], preferred_element_type=jnp.float32)
                          + bp2_ref[...])


def _row_spec(shape):
    nd = len(shape)
    if nd == 2:
        return pl.BlockSpec((R, shape[1]), lambda r: (r, 0))
    return pl.BlockSpec((shape[0], R, shape[2]), lambda r: (0, r, 0))


def _full_spec(shape):
    zeros = (0,) * len(shape)
    return pl.BlockSpec(shape, lambda r, _z=zeros: _z)


def _tc_call(body, in_arrays, row_flags, out_widths):
    in_specs = [
        _row_spec(a.shape) if rf else _full_spec(a.shape)
        for a, rf in zip(in_arrays, row_flags)
    ]
    out_shape = tuple(jax.ShapeDtypeStruct((N, wd), jnp.float32)
                      for wd in out_widths)
    out_specs = tuple(pl.BlockSpec((R, wd), lambda r, _w=wd: (r, 0))
                      for wd in out_widths)
    if len(out_widths) == 1:
        out_shape, out_specs = out_shape[0], out_specs[0]
    return pl.pallas_call(
        body,
        grid=(N // R,),
        in_specs=in_specs,
        out_specs=out_specs,
        out_shape=out_shape,
    )(*in_arrays)


def kernel(x, edge_index, W1, b1, W2, b2, Wm1, bm1, Wm2, bm2, Wp1, bp1, Wp2, bp2):
    npad_e = EPAD - E
    pad_src = jnp.asarray((np.arange(npad_e, dtype=np.int32) * 131) % N)
    pad_dst = jnp.asarray(N + (np.arange(npad_e, dtype=np.int32) % (NPAD - N)))
    src_flat = jnp.concatenate([edge_index[0], pad_src])
    dst_flat = jnp.concatenate([edge_index[1], pad_dst])
    dst3 = dst_flat.reshape(NS * CWIN, WR, B)
    zeros = jnp.zeros((NPAD, HH), jnp.float32)

    degp = _deg_kernel(dst3)
    degt = degp.reshape(NC, NPAD).T

    y1a, y1b = _tc_call(_tc1_body, (degt, x, W1), (True, True, False), (HH, HH))
    a1 = _conv_kernel(y1a, y1b, src_flat, dst3, zeros)
    h1, y2a, y2b = _tc_call(_tc2_body, (degt, a1, y1a, y1b, b1[None], W2),
                            (True, True, True, True, False, False), (H, HH, HH))
    a2 = _conv_kernel(y2a, y2b, src_flat, dst3, zeros)
    p = _tc_call(_tc3_body,
                 (degt, a2, y2a, y2b, h1, b2[None], Wm1, bm1[None], Wm2,
                  bm2[None], Wp1, bp1[None], Wp2, bp2[None]),
                 (True, True, True, True, True) + (False,) * 9, (H,))
    return p
